# baseline (speedup 1.0000x reference)
import jax
import jax.numpy as jnp
from jax.experimental import pallas as pl
from jax.experimental.pallas import tpu as pltpu

_C = 16
_N = 1048576
_W = 1024
_ROWS_PER_C = _N // _W
_J = 4
_BR = _ROWS_PER_C // _J


def _fold(x):
    acc = x[0:8, :]
    for k in range(1, _BR // 8):
        acc = acc + x[8 * k:8 * k + 8, :]
    out = acc[:, 0:128]
    for l in range(1, _W // 128):
        out = out + acc[:, 128 * l:128 * (l + 1)]
    return out


def _stage1(loss_ref, succ_ref, pt_ref, pa_ref):
    j = pl.program_id(1)
    x = loss_ref[...]
    masked = jnp.where(succ_ref[...] == 1, x, 0.0)
    pm = _fold(masked).reshape(1, 8, 128)
    px = _fold(x).reshape(1, 8, 128)

    @pl.when(j == 0)
    def _init():
        pt_ref[...] = pm
        pa_ref[...] = px

    @pl.when(j != 0)
    def _acc():
        pt_ref[...] += pm
        pa_ref[...] += px


def _stage2(pt_ref, pa_ref, out_ref):
    ts = jnp.sum(pt_ref[...], axis=1, keepdims=True)
    tt = jnp.sum(pa_ref[...], axis=1, keepdims=True)
    lv = jnp.log(ts / tt)
    kl = jnp.maximum(lv * lv - 0.01, 0.0)
    out_ref[...] = jnp.sum(kl, axis=0, keepdims=True)


def kernel(lossTensor, lcSuccesses):
    loss2 = lossTensor.reshape(_C * _ROWS_PER_C, _W)
    succ2 = lcSuccesses.reshape(_C * _ROWS_PER_C, _W)
    pt, pa = pl.pallas_call(
        _stage1,
        grid=(_C, _J),
        in_specs=[
            pl.BlockSpec((_BR, _W), lambda c, j: (c * _J + j, 0)),
            pl.BlockSpec((_BR, _W), lambda c, j: (c * _J + j, 0)),
        ],
        out_specs=[
            pl.BlockSpec((1, 8, 128), lambda c, j: (c, 0, 0)),
            pl.BlockSpec((1, 8, 128), lambda c, j: (c, 0, 0)),
        ],
        out_shape=[
            jax.ShapeDtypeStruct((_C, 8, 128), jnp.float32),
            jax.ShapeDtypeStruct((_C, 8, 128), jnp.float32),
        ],
        compiler_params=pltpu.CompilerParams(
            dimension_semantics=("arbitrary", "arbitrary"),
        ),
    )(loss2, succ2)

    out = pl.pallas_call(
        _stage2,
        in_specs=[
            pl.BlockSpec((_C, 8 * 128), lambda: (0, 0)),
            pl.BlockSpec((_C, 8 * 128), lambda: (0, 0)),
        ],
        out_specs=pl.BlockSpec((1, 1), lambda: (0, 0)),
        out_shape=jax.ShapeDtypeStruct((1, 1), jnp.float32),
    )(pt.reshape(_C, 8 * 128), pa.reshape(_C, 8 * 128))
    return out[0, 0]


# speedup vs baseline: 3.3834x; 3.3834x over previous
import jax
import jax.numpy as jnp
from jax.experimental import pallas as pl
from jax.experimental.pallas import tpu as pltpu

_C = 16
_N = 1048576
_BLK = 8192


def _fold(x):
    acc = x[:, 0:128]
    for l in range(1, _BLK // 128):
        acc = acc + x[:, 128 * l:128 * (l + 1)]
    return acc


def _body(loss_ref, succ_ref, out_ref, at_ref, aa_ref):
    i = pl.program_id(0)

    @pl.when(i == 0)
    def _init():
        at_ref[...] = jnp.zeros_like(at_ref)
        aa_ref[...] = jnp.zeros_like(aa_ref)

    x = loss_ref[...]
    masked = jnp.where(succ_ref[...] == 1, x, 0.0)
    at_ref[...] += _fold(masked)
    aa_ref[...] += _fold(x)

    @pl.when(i == pl.num_programs(0) - 1)
    def _fini():
        ts = jnp.sum(at_ref[...], axis=1, keepdims=True)
        tt = jnp.sum(aa_ref[...], axis=1, keepdims=True)
        lv = jnp.log(ts / tt)
        kl = jnp.maximum(lv * lv - 0.01, 0.0)
        out_ref[...] = jnp.sum(kl, axis=0, keepdims=True)


def kernel(lossTensor, lcSuccesses):
    grid = _N // _BLK
    out = pl.pallas_call(
        _body,
        grid=(grid,),
        in_specs=[
            pl.BlockSpec((_C, _BLK), lambda i: (0, i)),
            pl.BlockSpec((_C, _BLK), lambda i: (0, i)),
        ],
        out_specs=pl.BlockSpec((1, 1), lambda i: (0, 0)),
        out_shape=jax.ShapeDtypeStruct((1, 1), jnp.float32),
        scratch_shapes=[
            pltpu.VMEM((_C, 128), jnp.float32),
            pltpu.VMEM((_C, 128), jnp.float32),
        ],
        compiler_params=pltpu.CompilerParams(
            dimension_semantics=("arbitrary",),
        ),
    )(lossTensor, lcSuccesses)
    return out[0, 0]


# speedup vs baseline: 7.0948x; 2.0970x over previous
import jax
import jax.numpy as jnp
from jax.experimental import pallas as pl
from jax.experimental.pallas import tpu as pltpu

_C = 16
_N = 1048576
_BLK = 32768


def _fold(x):
    acc = x[:, 0:128]
    for l in range(1, _BLK // 128):
        acc = acc + x[:, 128 * l:128 * (l + 1)]
    return acc


def _body(loss_ref, succ_ref, out_ref, at_ref, aa_ref):
    i = pl.program_id(0)

    @pl.when(i == 0)
    def _init():
        at_ref[...] = jnp.zeros_like(at_ref)
        aa_ref[...] = jnp.zeros_like(aa_ref)

    x = loss_ref[...]
    masked = jnp.where(succ_ref[...] == 1, x, 0.0)
    at_ref[...] += _fold(masked)
    aa_ref[...] += _fold(x)

    @pl.when(i == pl.num_programs(0) - 1)
    def _fini():
        ts = jnp.sum(at_ref[...], axis=1, keepdims=True)
        tt = jnp.sum(aa_ref[...], axis=1, keepdims=True)
        lv = jnp.log(ts / tt)
        kl = jnp.maximum(lv * lv - 0.01, 0.0)
        out_ref[...] = jnp.sum(kl, axis=0, keepdims=True)


def kernel(lossTensor, lcSuccesses):
    grid = _N // _BLK
    out = pl.pallas_call(
        _body,
        grid=(grid,),
        in_specs=[
            pl.BlockSpec((_C, _BLK), lambda i: (0, i)),
            pl.BlockSpec((_C, _BLK), lambda i: (0, i)),
        ],
        out_specs=pl.BlockSpec((1, 1), lambda i: (0, 0)),
        out_shape=jax.ShapeDtypeStruct((1, 1), jnp.float32),
        scratch_shapes=[
            pltpu.VMEM((_C, 128), jnp.float32),
            pltpu.VMEM((_C, 128), jnp.float32),
        ],
        compiler_params=pltpu.CompilerParams(
            dimension_semantics=("arbitrary",),
        ),
    )(lossTensor, lcSuccesses)
    return out[0, 0]


# speedup vs baseline: 7.6360x; 1.0763x over previous
import jax
import jax.numpy as jnp
from jax.experimental import pallas as pl
from jax.experimental.pallas import tpu as pltpu

_C = 16
_N = 1048576
_BLK = 65536


def _fold(x):
    acc = x[:, 0:128]
    for l in range(1, _BLK // 128):
        acc = acc + x[:, 128 * l:128 * (l + 1)]
    return acc


def _body(loss_ref, succ_ref, out_ref, at_ref, aa_ref):
    i = pl.program_id(0)

    @pl.when(i == 0)
    def _init():
        at_ref[...] = jnp.zeros_like(at_ref)
        aa_ref[...] = jnp.zeros_like(aa_ref)

    x = loss_ref[...]
    masked = jnp.where(succ_ref[...] == 1, x, 0.0)
    at_ref[...] += _fold(masked)
    aa_ref[...] += _fold(x)

    @pl.when(i == pl.num_programs(0) - 1)
    def _fini():
        ts = jnp.sum(at_ref[...], axis=1, keepdims=True)
        tt = jnp.sum(aa_ref[...], axis=1, keepdims=True)
        lv = jnp.log(ts / tt)
        kl = jnp.maximum(lv * lv - 0.01, 0.0)
        out_ref[...] = jnp.sum(kl, axis=0, keepdims=True)


def kernel(lossTensor, lcSuccesses):
    grid = _N // _BLK
    out = pl.pallas_call(
        _body,
        grid=(grid,),
        in_specs=[
            pl.BlockSpec((_C, _BLK), lambda i: (0, i)),
            pl.BlockSpec((_C, _BLK), lambda i: (0, i)),
        ],
        out_specs=pl.BlockSpec((1, 1), lambda i: (0, 0)),
        out_shape=jax.ShapeDtypeStruct((1, 1), jnp.float32),
        scratch_shapes=[
            pltpu.VMEM((_C, 128), jnp.float32),
            pltpu.VMEM((_C, 128), jnp.float32),
        ],
        compiler_params=pltpu.CompilerParams(
            dimension_semantics=("arbitrary",),
        ),
    )(lossTensor, lcSuccesses)
    return out[0, 0]
